# Initial kernel scaffold; baseline (speedup 1.0000x reference)
import jax
import jax.numpy as jnp
from jax.experimental import pallas as pl


def kernel(x, w1, b1, w2, b2, wl1, bl1, wl2, bl2):
    raise NotImplementedError("write your pallas kernel here")


# speedup vs baseline: 15.9818x; 15.9818x over previous
import functools

import jax
import jax.numpy as jnp
from jax.experimental import pallas as pl
from jax.experimental.pallas import tpu as pltpu


def _cdiv(a, b):
    return (a + b - 1) // b


def _fused_cnn_kernel(x_ref, w1s_ref, b1_ref, w2s_ref, b2_ref,
                      wl1_ref, bl1_ref, wl2_ref, bl2_ref, o_ref):
    bt = x_ref.shape[0]
    xp = x_ref[...]

    lhs1 = jnp.concatenate([xp[:, 0:16, :], xp[:, 1:17, :]], axis=2)
    lhs1 = lhs1.reshape(bt * 16, 112)
    a1 = jnp.dot(lhs1, w1s_ref[...],
                 preferred_element_type=jnp.float32)
    a1 = jnp.maximum(a1 + b1_ref[...], 0.0).astype(jnp.bfloat16)
    p1 = jnp.maximum(
        jnp.maximum(a1[:, 0:256], a1[:, 256:512]),
        jnp.maximum(a1[:, 512:768], a1[:, 768:1024]))

    p13 = p1.reshape(bt, 16, 256)
    pad = jnp.zeros((bt, 2, 256), jnp.bfloat16)
    p1p = jnp.concatenate([p13, pad], axis=1)
    lhs2 = jnp.concatenate(
        [p1p[:, 0:16, :], p1p[:, 1:17, :], p1p[:, 2:18, :]], axis=2)
    lhs2 = lhs2.reshape(bt * 16, 768)
    a2 = jnp.dot(lhs2, w2s_ref[...],
                 preferred_element_type=jnp.float32)
    a2 = jnp.maximum(a2 + b2_ref[...], 0.0).astype(jnp.bfloat16)

    m2 = jnp.maximum(a2[:, 0:256], a2[:, 256:512])
    p2 = m2.reshape(bt, 8, 2, 256).max(axis=2)

    flat = jnp.concatenate([p2[:, h, :] for h in range(5)],
                           axis=1)
    h1 = jnp.dot(flat, wl1_ref[...],
                 preferred_element_type=jnp.float32)
    h1 = jnp.maximum(h1 + bl1_ref[...], 0.0).astype(jnp.bfloat16)
    lg = jnp.dot(h1, wl2_ref[...],
                 preferred_element_type=jnp.float32) + bl2_ref[...]

    mx = jnp.max(lg, axis=1, keepdims=True)
    lse = jnp.log(jnp.sum(jnp.exp(lg - mx), axis=1, keepdims=True)) + mx
    o_ref[...] = lg - lse


def _build_weights(w1, b1, w2, b2, wl1, bl1, wl2, bl2):
    f32 = jnp.float32
    w1s = jnp.zeros((2, 2, 28, 2, 2, 16, 16), f32)
    wpv = jnp.arange(13)
    for kh in range(3):
        for kw in range(3):
            for h_par in range(2):
                dh = h_par + kh
                ro, q = divmod(dh, 2)
                for w_par in range(2):
                    win = 2 * wpv + w_par + kw
                    w1s = w1s.at[ro, q, win, h_par, w_par, wpv, :].set(
                        jnp.broadcast_to(w1[kh, kw, 0, :], (13, 16)))
    w1s = w1s.reshape(112, 1024).astype(jnp.bfloat16)
    blk = jnp.concatenate([jnp.tile(b1.reshape(16), 13), jnp.zeros(48, f32)])
    b1row = jnp.tile(blk, 4).reshape(1, 1024)

    w2s = jnp.zeros((3, 16, 16, 2, 8, 32), f32)
    wp2v = jnp.arange(5)
    for kh in range(3):
        for kw in range(3):
            for w_par in range(2):
                w2s = w2s.at[kh, 2 * wp2v + w_par + kw, :, w_par, wp2v, :].set(
                    jnp.broadcast_to(w2[kh, kw, :, :], (5, 16, 32)))
    w2s = w2s.reshape(768, 512).astype(jnp.bfloat16)
    blk2 = jnp.concatenate([jnp.tile(b2.reshape(32), 5), jnp.zeros(96, f32)])
    b2row = jnp.tile(blk2, 2).reshape(1, 512)

    wl1r = wl1.reshape(32, 5, 5, 128).transpose(1, 2, 0, 3)
    wl1p = jnp.zeros((5, 8, 32, 128), f32).at[:, :5, :, :].set(wl1r)
    wl1p = wl1p.reshape(1280, 128).astype(jnp.bfloat16)

    return (w1s, b1row, w2s, b2row, wl1p,
            bl1.reshape(1, 128).astype(f32),
            wl2.astype(jnp.bfloat16), bl2.reshape(1, 10).astype(f32))


@functools.partial(jax.jit, static_argnames=("batch_tile",))
def _forward(x, w1, b1, w2, b2, wl1, bl1, wl2, bl2, batch_tile=128):
    B = x.shape[0]
    xb = x.astype(jnp.bfloat16).reshape(B, 14, 56)
    xb = jnp.pad(xb, ((0, 0), (0, 3), (0, 0)))
    bt = min(batch_tile, B)
    bp = _cdiv(B, bt) * bt
    if bp != B:
        xb = jnp.pad(xb, ((0, bp - B), (0, 0), (0, 0)))

    packed = _build_weights(w1, b1, w2, b2, wl1, bl1, wl2, bl2)
    w1s, b1row, w2s, b2row, wl1p, bl1r, wl2b, bl2r = packed

    out = pl.pallas_call(
        _fused_cnn_kernel,
        out_shape=jax.ShapeDtypeStruct((bp, 10), jnp.float32),
        grid=(bp // bt,),
        in_specs=[
            pl.BlockSpec((bt, 17, 56), lambda b: (b, 0, 0)),
            pl.BlockSpec((112, 1024), lambda b: (0, 0)),
            pl.BlockSpec((1, 1024), lambda b: (0, 0)),
            pl.BlockSpec((768, 512), lambda b: (0, 0)),
            pl.BlockSpec((1, 512), lambda b: (0, 0)),
            pl.BlockSpec((1280, 128), lambda b: (0, 0)),
            pl.BlockSpec((1, 128), lambda b: (0, 0)),
            pl.BlockSpec((128, 10), lambda b: (0, 0)),
            pl.BlockSpec((1, 10), lambda b: (0, 0)),
        ],
        out_specs=pl.BlockSpec((bt, 10), lambda b: (b, 0)),
        compiler_params=pltpu.CompilerParams(
            dimension_semantics=("parallel",)),
    )(xb, w1s, b1row, w2s, b2row, wl1p, bl1r, wl2b, bl2r)
    return out[:B]


def kernel(x, w1, b1, w2, b2, wl1, bl1, wl2, bl2):
    return _forward(x, w1, b1, w2, b2, wl1, bl1, wl2, bl2)


# speedup vs baseline: 17.0799x; 1.0687x over previous
import functools

import jax
import jax.numpy as jnp
import numpy as np
from jax.experimental import pallas as pl
from jax.experimental.pallas import tpu as pltpu


def _conv1_pattern():
    m = np.zeros((112, 64, 9), np.float32)
    for kh in range(3):
        for kw in range(3):
            for h_par in range(2):
                ro, q = divmod(h_par + kh, 2)
                for w_par in range(2):
                    for wp in range(13):
                        win = 2 * wp + w_par + kw
                        m[ro * 56 + q * 28 + win,
                          (h_par * 2 + w_par) * 16 + wp, kh * 3 + kw] = 1.0
    return m


def _conv2_pattern():
    m = np.zeros((3, 16, 16, 3), np.float32)
    for kh in range(3):
        for kw in range(3):
            for w_par in range(2):
                for wp2 in range(5):
                    m[kh, 2 * wp2 + w_par + kw, w_par * 8 + wp2, kw] = 1.0
    return m


def _bias1_pattern():
    s = np.zeros((1024, 16), np.float32)
    for blk in range(4):
        for wp in range(13):
            for c in range(16):
                s[(blk * 16 + wp) * 16 + c, c] = 1.0
    return s


def _bias2_pattern():
    s = np.zeros((512, 32), np.float32)
    for w_par in range(2):
        for wp2 in range(5):
            for co in range(32):
                s[(w_par * 8 + wp2) * 32 + co, co] = 1.0
    return s


_M1 = _conv1_pattern()
_M2 = _conv2_pattern()
_S1 = _bias1_pattern()
_S2 = _bias2_pattern()


def _cdiv(a, b):
    return (a + b - 1) // b


def _fused_cnn_kernel(x_ref, w1s_ref, b1_ref, w2s_ref, b2_ref,
                      wl1_ref, bl1_ref, wl2_ref, bl2_ref, o_ref):
    bt = x_ref.shape[0]
    xr = x_ref[...].astype(jnp.bfloat16)
    xp = jnp.concatenate(
        [xr, jnp.zeros((bt, 3, 56), jnp.bfloat16)], axis=1)

    lhs1 = jnp.concatenate([xp[:, 0:16, :], xp[:, 1:17, :]], axis=2)
    lhs1 = lhs1.reshape(bt * 16, 112)
    a1 = jnp.dot(lhs1, w1s_ref[...],
                 preferred_element_type=jnp.float32)
    a1 = jnp.maximum(a1 + b1_ref[...], 0.0).astype(jnp.bfloat16)
    p1 = jnp.maximum(
        jnp.maximum(a1[:, 0:256], a1[:, 256:512]),
        jnp.maximum(a1[:, 512:768], a1[:, 768:1024]))

    p13 = p1.reshape(bt, 16, 256)
    pad = jnp.zeros((bt, 2, 256), jnp.bfloat16)
    p1p = jnp.concatenate([p13, pad], axis=1)
    lhs2 = jnp.concatenate(
        [p1p[:, 0:16, :], p1p[:, 1:17, :], p1p[:, 2:18, :]], axis=2)
    lhs2 = lhs2.reshape(bt * 16, 768)
    a2 = jnp.dot(lhs2, w2s_ref[...],
                 preferred_element_type=jnp.float32)
    a2 = jnp.maximum(a2 + b2_ref[...], 0.0).astype(jnp.bfloat16)

    m2 = jnp.maximum(a2[:, 0:256], a2[:, 256:512])
    p2 = m2.reshape(bt, 8, 2, 256).max(axis=2)

    flat = jnp.concatenate([p2[:, h, :] for h in range(5)],
                           axis=1)
    h1 = jnp.dot(flat, wl1_ref[...],
                 preferred_element_type=jnp.float32)
    h1 = jnp.maximum(h1 + bl1_ref[...], 0.0).astype(jnp.bfloat16)
    lg = jnp.dot(h1, wl2_ref[...],
                 preferred_element_type=jnp.float32) + bl2_ref[...]

    mx = jnp.max(lg, axis=1, keepdims=True)
    lse = jnp.log(jnp.sum(jnp.exp(lg - mx), axis=1, keepdims=True)) + mx
    o_ref[...] = lg - lse


def _build_weights(w1, b1, w2, b2, wl1, bl1, wl2, bl2):
    f32 = jnp.float32
    w1s = jnp.einsum("knt,tc->knc", _M1, w1.reshape(9, 16))
    w1s = w1s.reshape(112, 1024).astype(jnp.bfloat16)
    b1row = (_S1 @ b1.reshape(16)).reshape(1, 1024)

    w2s = jnp.einsum("hwpk,hkio->hwipo", _M2, w2)
    w2s = w2s.reshape(768, 512).astype(jnp.bfloat16)
    b2row = (_S2 @ b2.reshape(32)).reshape(1, 512)

    wl1r = wl1.reshape(32, 5, 5, 128).transpose(1, 2, 0, 3)
    wl1p = jnp.pad(wl1r, ((0, 0), (0, 3), (0, 0), (0, 0)))
    wl1p = wl1p.reshape(1280, 128).astype(jnp.bfloat16)

    return (w1s, b1row, w2s, b2row, wl1p,
            bl1.reshape(1, 128).astype(f32),
            wl2.astype(jnp.bfloat16), bl2.reshape(1, 10).astype(f32))


@functools.partial(jax.jit, static_argnames=("batch_tile",))
def _forward(x, w1, b1, w2, b2, wl1, bl1, wl2, bl2, batch_tile=128):
    B = x.shape[0]
    xb = x.reshape(B, 14, 56)
    bt = min(batch_tile, B)
    bp = _cdiv(B, bt) * bt
    if bp != B:
        xb = jnp.pad(xb, ((0, bp - B), (0, 0), (0, 0)))

    packed = _build_weights(w1, b1, w2, b2, wl1, bl1, wl2, bl2)
    w1s, b1row, w2s, b2row, wl1p, bl1r, wl2b, bl2r = packed

    out = pl.pallas_call(
        _fused_cnn_kernel,
        out_shape=jax.ShapeDtypeStruct((bp, 10), jnp.float32),
        grid=(bp // bt,),
        in_specs=[
            pl.BlockSpec((bt, 14, 56), lambda b: (b, 0, 0)),
            pl.BlockSpec((112, 1024), lambda b: (0, 0)),
            pl.BlockSpec((1, 1024), lambda b: (0, 0)),
            pl.BlockSpec((768, 512), lambda b: (0, 0)),
            pl.BlockSpec((1, 512), lambda b: (0, 0)),
            pl.BlockSpec((1280, 128), lambda b: (0, 0)),
            pl.BlockSpec((1, 128), lambda b: (0, 0)),
            pl.BlockSpec((128, 10), lambda b: (0, 0)),
            pl.BlockSpec((1, 10), lambda b: (0, 0)),
        ],
        out_specs=pl.BlockSpec((bt, 10), lambda b: (b, 0)),
        compiler_params=pltpu.CompilerParams(
            dimension_semantics=("parallel",)),
    )(xb, w1s, b1row, w2s, b2row, wl1p, bl1r, wl2b, bl2r)
    return out[:B]


def kernel(x, w1, b1, w2, b2, wl1, bl1, wl2, bl2):
    return _forward(x, w1, b1, w2, b2, wl1, bl1, wl2, bl2)


# speedup vs baseline: 18.0054x; 1.0542x over previous
import functools

import jax
import jax.numpy as jnp
import numpy as np
from jax.experimental import pallas as pl
from jax.experimental.pallas import tpu as pltpu


def _conv1_pattern():
    m = np.zeros((112, 64, 9), np.float32)
    for kh in range(3):
        for kw in range(3):
            for h_par in range(2):
                ro, q = divmod(h_par + kh, 2)
                for w_par in range(2):
                    for wp in range(13):
                        win = 2 * wp + w_par + kw
                        m[ro * 56 + q * 28 + win,
                          (h_par * 2 + w_par) * 16 + wp, kh * 3 + kw] = 1.0
    return m


def _conv2_pattern():
    m = np.zeros((3, 16, 16, 3), np.float32)
    for kh in range(3):
        for kw in range(3):
            for w_par in range(2):
                for wp2 in range(5):
                    m[kh, 2 * wp2 + w_par + kw, w_par * 8 + wp2, kw] = 1.0
    return m


def _bias1_pattern():
    s = np.zeros((1024, 16), np.float32)
    for blk in range(4):
        for wp in range(13):
            for c in range(16):
                s[(blk * 16 + wp) * 16 + c, c] = 1.0
    return s


def _bias2_pattern():
    s = np.zeros((512, 32), np.float32)
    for w_par in range(2):
        for wp2 in range(5):
            for co in range(32):
                s[(w_par * 8 + wp2) * 32 + co, co] = 1.0
    return s


_M1 = _conv1_pattern()
_M2 = _conv2_pattern()
_S1 = _bias1_pattern()
_S2 = _bias2_pattern()


def _cdiv(a, b):
    return (a + b - 1) // b


def _fused_cnn_kernel(x_ref, w1s_ref, b1_ref, w2s_ref, b2_ref,
                      wl1_ref, bl1_ref, wl2_ref, bl2_ref, o_ref):
    bt = x_ref.shape[0]
    xr = x_ref[...].astype(jnp.bfloat16)
    xp = jnp.concatenate(
        [xr, jnp.zeros((bt, 3, 56), jnp.bfloat16)], axis=1)

    lhs1 = jnp.concatenate([xp[:, 0:16, :], xp[:, 1:17, :]], axis=2)
    lhs1 = lhs1.reshape(bt * 16, 112)
    a1 = jnp.dot(lhs1, w1s_ref[...],
                 preferred_element_type=jnp.float32)
    a1 = jnp.maximum(a1 + b1_ref[...], 0.0).astype(jnp.bfloat16)
    p1 = jnp.maximum(
        jnp.maximum(a1[:, 0:256], a1[:, 256:512]),
        jnp.maximum(a1[:, 512:768], a1[:, 768:1024]))

    p13 = p1.reshape(bt, 16, 256)
    pad = jnp.zeros((bt, 2, 256), jnp.bfloat16)
    p1p = jnp.concatenate([p13, pad], axis=1)
    lhs2 = jnp.concatenate(
        [p1p[:, 0:16, :], p1p[:, 1:17, :], p1p[:, 2:18, :]], axis=2)
    lhs2 = lhs2.reshape(bt * 16, 768)
    a2 = jnp.dot(lhs2, w2s_ref[...],
                 preferred_element_type=jnp.float32)
    a2 = jnp.maximum(a2 + b2_ref[...], 0.0).astype(jnp.bfloat16)

    m2 = jnp.maximum(a2[:, 0:256], a2[:, 256:512])
    p2 = m2.reshape(bt, 8, 2, 256).max(axis=2)

    flat = jnp.concatenate([p2[:, h, :] for h in range(5)],
                           axis=1)
    h1 = jnp.dot(flat, wl1_ref[...],
                 preferred_element_type=jnp.float32)
    h1 = jnp.maximum(h1 + bl1_ref[...], 0.0).astype(jnp.bfloat16)
    lg = jnp.dot(h1, wl2_ref[...],
                 preferred_element_type=jnp.float32) + bl2_ref[...]

    mx = jnp.max(lg, axis=1, keepdims=True)
    lse = jnp.log(jnp.sum(jnp.exp(lg - mx), axis=1, keepdims=True)) + mx
    o_ref[...] = lg - lse


def _build_weights(w1, b1, w2, b2, wl1, bl1, wl2, bl2):
    f32 = jnp.float32
    w1s = jnp.einsum("knt,tc->knc", _M1, w1.reshape(9, 16))
    w1s = w1s.reshape(112, 1024).astype(jnp.bfloat16)
    b1row = (_S1 @ b1.reshape(16)).reshape(1, 1024)

    w2s = jnp.einsum("hwpk,hkio->hwipo", _M2, w2)
    w2s = w2s.reshape(768, 512).astype(jnp.bfloat16)
    b2row = (_S2 @ b2.reshape(32)).reshape(1, 512)

    wl1r = wl1.reshape(32, 5, 5, 128).transpose(1, 2, 0, 3)
    wl1p = jnp.pad(wl1r, ((0, 0), (0, 3), (0, 0), (0, 0)))
    wl1p = wl1p.reshape(1280, 128).astype(jnp.bfloat16)

    return (w1s, b1row, w2s, b2row, wl1p,
            bl1.reshape(1, 128).astype(f32),
            wl2.astype(jnp.bfloat16), bl2.reshape(1, 10).astype(f32))


@functools.partial(jax.jit, static_argnames=("batch_tile",))
def _forward(x, w1, b1, w2, b2, wl1, bl1, wl2, bl2, batch_tile=128):
    B = x.shape[0]
    xb = x.reshape(B, 14, 56)
    bt = min(batch_tile, B)
    bp = _cdiv(B, bt) * bt
    if bp != B:
        xb = jnp.pad(xb, ((0, bp - B), (0, 0), (0, 0)))

    packed = _build_weights(w1, b1, w2, b2, wl1, bl1, wl2, bl2)
    w1s, b1row, w2s, b2row, wl1p, bl1r, wl2b, bl2r = packed

    out = pl.pallas_call(
        _fused_cnn_kernel,
        out_shape=jax.ShapeDtypeStruct((bp, 10), jnp.float32),
        grid=(bp // bt,),
        in_specs=[
            pl.BlockSpec((bt, 14, 56), lambda b: (b, 0, 0)),
            pl.BlockSpec((112, 1024), lambda b: (0, 0)),
            pl.BlockSpec((1, 1024), lambda b: (0, 0)),
            pl.BlockSpec((768, 512), lambda b: (0, 0)),
            pl.BlockSpec((1, 512), lambda b: (0, 0)),
            pl.BlockSpec((1280, 128), lambda b: (0, 0)),
            pl.BlockSpec((1, 128), lambda b: (0, 0)),
            pl.BlockSpec((128, 10), lambda b: (0, 0)),
            pl.BlockSpec((1, 10), lambda b: (0, 0)),
        ],
        out_specs=pl.BlockSpec((bt, 10), lambda b: (b, 0)),
        compiler_params=pltpu.CompilerParams(
            dimension_semantics=("parallel",)),
    )(xb, w1s, b1row, w2s, b2row, wl1p, bl1r, wl2b, bl2r)
    return out[:B]


def kernel(x, w1, b1, w2, b2, wl1, bl1, wl2, bl2):
    return _forward(x, w1, b1, w2, b2, wl1, bl1, wl2, bl2, batch_tile=256)


# speedup vs baseline: 18.4468x; 1.0245x over previous
import functools

import jax
import jax.numpy as jnp
import numpy as np
from jax.experimental import pallas as pl
from jax.experimental.pallas import tpu as pltpu
from jax.experimental.shard_map import shard_map
from jax.sharding import Mesh, PartitionSpec as P


def _conv1_pattern():
    m = np.zeros((112, 64, 9), np.float32)
    for kh in range(3):
        for kw in range(3):
            for h_par in range(2):
                ro, q = divmod(h_par + kh, 2)
                for w_par in range(2):
                    for wp in range(13):
                        win = 2 * wp + w_par + kw
                        m[ro * 56 + q * 28 + win,
                          (h_par * 2 + w_par) * 16 + wp, kh * 3 + kw] = 1.0
    return m


def _conv2_pattern():
    m = np.zeros((3, 16, 16, 3), np.float32)
    for kh in range(3):
        for kw in range(3):
            for w_par in range(2):
                for wp2 in range(5):
                    m[kh, 2 * wp2 + w_par + kw, w_par * 8 + wp2, kw] = 1.0
    return m


def _bias1_pattern():
    s = np.zeros((1024, 16), np.float32)
    for blk in range(4):
        for wp in range(13):
            for c in range(16):
                s[(blk * 16 + wp) * 16 + c, c] = 1.0
    return s


def _bias2_pattern():
    s = np.zeros((512, 32), np.float32)
    for w_par in range(2):
        for wp2 in range(5):
            for co in range(32):
                s[(w_par * 8 + wp2) * 32 + co, co] = 1.0
    return s


_M1 = _conv1_pattern()
_M2 = _conv2_pattern()
_S1 = _bias1_pattern()
_S2 = _bias2_pattern()


def _cdiv(a, b):
    return (a + b - 1) // b


def _fused_cnn_kernel(x_ref, w1s_ref, b1_ref, w2s_ref, b2_ref,
                      wl1_ref, bl1_ref, wl2_ref, bl2_ref, o_ref):
    bt = x_ref.shape[0]
    xr = x_ref[...].astype(jnp.bfloat16)
    xp = jnp.concatenate(
        [xr, jnp.zeros((bt, 3, 56), jnp.bfloat16)], axis=1)

    lhs1 = jnp.concatenate([xp[:, 0:16, :], xp[:, 1:17, :]], axis=2)
    lhs1 = lhs1.reshape(bt * 16, 112)
    a1 = jnp.dot(lhs1, w1s_ref[...],
                 preferred_element_type=jnp.float32)
    a1 = jnp.maximum(a1 + b1_ref[...], 0.0).astype(jnp.bfloat16)
    p1 = jnp.maximum(
        jnp.maximum(a1[:, 0:256], a1[:, 256:512]),
        jnp.maximum(a1[:, 512:768], a1[:, 768:1024]))

    p13 = p1.reshape(bt, 16, 256)
    pad = jnp.zeros((bt, 2, 256), jnp.bfloat16)
    p1p = jnp.concatenate([p13, pad], axis=1)
    lhs2 = jnp.concatenate(
        [p1p[:, 0:16, :], p1p[:, 1:17, :], p1p[:, 2:18, :]], axis=2)
    lhs2 = lhs2.reshape(bt * 16, 768)
    a2 = jnp.dot(lhs2, w2s_ref[...],
                 preferred_element_type=jnp.float32)
    a2 = jnp.maximum(a2 + b2_ref[...], 0.0).astype(jnp.bfloat16)

    m2 = jnp.maximum(a2[:, 0:256], a2[:, 256:512])
    p2 = m2.reshape(bt, 8, 2, 256).max(axis=2)

    flat = jnp.concatenate([p2[:, h, :] for h in range(5)],
                           axis=1)
    h1 = jnp.dot(flat, wl1_ref[...],
                 preferred_element_type=jnp.float32)
    h1 = jnp.maximum(h1 + bl1_ref[...], 0.0).astype(jnp.bfloat16)
    lg = jnp.dot(h1, wl2_ref[...],
                 preferred_element_type=jnp.float32) + bl2_ref[...]

    mx = jnp.max(lg, axis=1, keepdims=True)
    lse = jnp.log(jnp.sum(jnp.exp(lg - mx), axis=1, keepdims=True)) + mx
    o_ref[...] = lg - lse


def _build_weights(w1, b1, w2, b2, wl1, bl1, wl2, bl2):
    f32 = jnp.float32
    w1s = jnp.einsum("knt,tc->knc", _M1, w1.reshape(9, 16))
    w1s = w1s.reshape(112, 1024).astype(jnp.bfloat16)
    b1row = (_S1 @ b1.reshape(16)).reshape(1, 1024)

    w2s = jnp.einsum("hwpk,hkio->hwipo", _M2, w2)
    w2s = w2s.reshape(768, 512).astype(jnp.bfloat16)
    b2row = (_S2 @ b2.reshape(32)).reshape(1, 512)

    wl1r = wl1.reshape(32, 5, 5, 128).transpose(1, 2, 0, 3)
    wl1p = jnp.pad(wl1r, ((0, 0), (0, 3), (0, 0), (0, 0)))
    wl1p = wl1p.reshape(1280, 128).astype(jnp.bfloat16)

    return (w1s, b1row, w2s, b2row, wl1p,
            bl1.reshape(1, 128).astype(f32),
            wl2.astype(jnp.bfloat16), bl2.reshape(1, 10).astype(f32))


def _pallas_forward(xb, w1s, b1row, w2s, b2row, wl1p, bl1r, wl2b, bl2r,
                    *, batch_tile):
    bp = xb.shape[0]
    bt = min(batch_tile, bp)
    return pl.pallas_call(
        _fused_cnn_kernel,
        out_shape=jax.ShapeDtypeStruct((bp, 10), jnp.float32),
        grid=(bp // bt,),
        in_specs=[
            pl.BlockSpec((bt, 14, 56), lambda b: (b, 0, 0)),
            pl.BlockSpec((112, 1024), lambda b: (0, 0)),
            pl.BlockSpec((1, 1024), lambda b: (0, 0)),
            pl.BlockSpec((768, 512), lambda b: (0, 0)),
            pl.BlockSpec((1, 512), lambda b: (0, 0)),
            pl.BlockSpec((1280, 128), lambda b: (0, 0)),
            pl.BlockSpec((1, 128), lambda b: (0, 0)),
            pl.BlockSpec((128, 10), lambda b: (0, 0)),
            pl.BlockSpec((1, 10), lambda b: (0, 0)),
        ],
        out_specs=pl.BlockSpec((bt, 10), lambda b: (b, 0)),
        compiler_params=pltpu.CompilerParams(
            dimension_semantics=("parallel",)),
    )(xb, w1s, b1row, w2s, b2row, wl1p, bl1r, wl2b, bl2r)


_DEVICES = jax.devices()
_NDEV = 2 if len(_DEVICES) >= 2 else 1
_MESH = Mesh(np.array(_DEVICES[:_NDEV]), ("d",)) if _NDEV > 1 else None


@functools.partial(jax.jit, static_argnames=("batch_tile",))
def _forward(x, w1, b1, w2, b2, wl1, bl1, wl2, bl2, batch_tile=256):
    B = x.shape[0]
    xb = x.reshape(B, 14, 56)
    bt = min(batch_tile, B)
    bp = _cdiv(B, bt) * bt
    if bp != B:
        xb = jnp.pad(xb, ((0, bp - B), (0, 0), (0, 0)))

    packed = _build_weights(w1, b1, w2, b2, wl1, bl1, wl2, bl2)
    run = functools.partial(_pallas_forward, batch_tile=bt)
    if _MESH is not None and bp % (_NDEV * bt) == 0:
        wspecs = tuple(P(None, None) for _ in range(8))
        run = shard_map(run, mesh=_MESH,
                        in_specs=(P("d", None, None),) + wspecs,
                        out_specs=P("d", None), check_rep=False)
    out = run(xb, *packed)
    return out[:B]


def kernel(x, w1, b1, w2, b2, wl1, bl1, wl2, bl2):
    return _forward(x, w1, b1, w2, b2, wl1, bl1, wl2, bl2)


# speedup vs baseline: 20.8217x; 1.1287x over previous
import functools

import jax
import jax.numpy as jnp
import numpy as np
from jax.experimental import pallas as pl
from jax.experimental.pallas import tpu as pltpu
from jax.experimental.shard_map import shard_map
from jax.sharding import Mesh, PartitionSpec as P


def _cdiv(a, b):
    return (a + b - 1) // b


def _conv1_pattern():
    m = np.zeros((224, 128, 9), np.float32)
    for kh in range(3):
        for kw in range(3):
            for hp_par in range(2):
                for h_par in range(2):
                    dh = 2 * hp_par + h_par + kh
                    ro, d4 = divmod(dh, 4)
                    for w_par in range(2):
                        for wp in range(13):
                            win = 2 * wp + w_par + kw
                            m[ro * 112 + d4 * 28 + win,
                              ((hp_par * 2 + h_par) * 2 + w_par) * 16 + wp,
                              kh * 3 + kw] = 1.0
    return m


def _conv2_pattern():
    m = np.zeros((80, 32, 9), np.float32)
    for kh in range(3):
        for kw in range(3):
            for h_par2 in range(2):
                dh2 = h_par2 + kh
                for w_par in range(2):
                    for wp2 in range(5):
                        wp = 2 * wp2 + w_par + kw
                        m[dh2 * 16 + wp,
                          (h_par2 * 2 + w_par) * 8 + wp2,
                          kh * 3 + kw] = 1.0
    return m


def _bias1_pattern():
    s = np.zeros((2048, 16), np.float32)
    for blk in range(8):
        for wp in range(13):
            for c in range(16):
                s[(blk * 16 + wp) * 16 + c, c] = 1.0
    return s


def _bias2_pattern():
    s = np.zeros((1024, 32), np.float32)
    for blk in range(4):
        for wp2 in range(5):
            for co in range(32):
                s[(blk * 8 + wp2) * 32 + co, co] = 1.0
    return s


_M1 = _conv1_pattern()
_M2 = _conv2_pattern()
_S1 = _bias1_pattern()
_S2 = _bias2_pattern()
_E240 = np.zeros(2048, np.float32)
_E240[240] = 1.0
_E240K = np.zeros(1280, np.float32)
_E240K[240] = 1.0


def _fused_cnn_kernel(x_ref, w1s_ref, w2s_ref,
                      wl1_ref, bl1_ref, wl2_ref, bl2_ref, o_ref):
    bt = x_ref.shape[0]
    xr = x_ref[...]
    xp = jnp.concatenate(
        [xr, jnp.zeros((bt, 2, 112), jnp.bfloat16)], axis=1)

    lhs1 = jnp.concatenate(
        [xp[:, 0:8, :], xp[:, 1:9, :],
         jnp.ones((bt, 8, 1), jnp.bfloat16)], axis=2)
    lhs1 = lhs1.reshape(bt * 8, 225)
    a1 = jnp.dot(lhs1, w1s_ref[...],
                 preferred_element_type=jnp.float32)
    a1 = jnp.maximum(a1, 0.0).astype(jnp.bfloat16)
    p1e = jnp.maximum(
        jnp.maximum(a1[:, 0:256], a1[:, 256:512]),
        jnp.maximum(a1[:, 512:768], a1[:, 768:1024]))
    p1o = jnp.maximum(
        jnp.maximum(a1[:, 1024:1280], a1[:, 1280:1536]),
        jnp.maximum(a1[:, 1536:1792], a1[:, 1792:2048]))
    p1 = jnp.concatenate([p1e, p1o], axis=1)

    p13 = p1.reshape(bt, 8, 512)
    p1p = jnp.concatenate(
        [p13, jnp.zeros((bt, 2, 512), jnp.bfloat16)], axis=1)
    lhs2 = jnp.concatenate(
        [p1p[:, 0:8, :], p1p[:, 1:9, :], p1p[:, 2:10, 0:256]], axis=2)
    lhs2 = lhs2.reshape(bt * 8, 1280)
    a2 = jnp.dot(lhs2, w2s_ref[...],
                 preferred_element_type=jnp.float32)
    a2 = jnp.maximum(a2, 0.0)
    p2 = jnp.maximum(
        jnp.maximum(a2[:, 0:256], a2[:, 256:512]),
        jnp.maximum(a2[:, 512:768], a2[:, 768:1024]))

    p2u = p2.reshape(bt, 8, 256)
    flat = jnp.concatenate([p2u[:, h, :] for h in range(5)],
                           axis=1).astype(jnp.bfloat16)
    h1 = jnp.dot(flat, wl1_ref[...],
                 preferred_element_type=jnp.float32)
    h1 = jnp.maximum(h1 + bl1_ref[...], 0.0).astype(jnp.bfloat16)
    lg = jnp.dot(h1, wl2_ref[...],
                 preferred_element_type=jnp.float32) + bl2_ref[...]

    mx = jnp.max(lg, axis=1, keepdims=True)
    lse = jnp.log(jnp.sum(jnp.exp(lg - mx), axis=1, keepdims=True)) + mx
    o_ref[...] = lg - lse


def _build_weights(w1, b1, w2, b2, wl1, bl1, wl2, bl2):
    f32 = jnp.float32
    w1s = jnp.einsum("knt,tc->knc", _M1, w1.reshape(9, 16))
    b1row = (_S1 @ b1.reshape(16) + _E240).reshape(1, 2048)
    w1s = jnp.concatenate([w1s.reshape(224, 2048), b1row])
    w1s = w1s.astype(jnp.bfloat16)

    w2s = jnp.einsum("rnt,tio->rino", _M2, w2.reshape(9, 16, 32))
    w2s = w2s.reshape(1280, 1024)
    w2s = w2s + _E240K[:, None] * (_S2 @ b2.reshape(32))[None, :]
    w2s = w2s.astype(jnp.bfloat16)

    wl1r = wl1.reshape(32, 5, 5, 128).transpose(1, 2, 0, 3)
    wl1p = jnp.pad(wl1r, ((0, 0), (0, 3), (0, 0), (0, 0)))
    wl1p = wl1p.reshape(1280, 128).astype(jnp.bfloat16)

    return (w1s, w2s, wl1p,
            bl1.reshape(1, 128).astype(f32),
            wl2.astype(jnp.bfloat16), bl2.reshape(1, 10).astype(f32))


def _pallas_forward(xb, w1s, w2s, wl1p, bl1r, wl2b, bl2r, *, batch_tile):
    bp = xb.shape[0]
    bt = min(batch_tile, bp)
    return pl.pallas_call(
        _fused_cnn_kernel,
        out_shape=jax.ShapeDtypeStruct((bp, 10), jnp.float32),
        grid=(bp // bt,),
        in_specs=[
            pl.BlockSpec((bt, 7, 112), lambda b: (b, 0, 0)),
            pl.BlockSpec((225, 2048), lambda b: (0, 0)),
            pl.BlockSpec((1280, 1024), lambda b: (0, 0)),
            pl.BlockSpec((1280, 128), lambda b: (0, 0)),
            pl.BlockSpec((1, 128), lambda b: (0, 0)),
            pl.BlockSpec((128, 10), lambda b: (0, 0)),
            pl.BlockSpec((1, 10), lambda b: (0, 0)),
        ],
        out_specs=pl.BlockSpec((bt, 10), lambda b: (b, 0)),
        compiler_params=pltpu.CompilerParams(
            dimension_semantics=("parallel",)),
    )(xb, w1s, w2s, wl1p, bl1r, wl2b, bl2r)


_DEVICES = jax.devices()
_NDEV = 2 if len(_DEVICES) >= 2 else 1
_MESH = Mesh(np.array(_DEVICES[:_NDEV]), ("d",)) if _NDEV > 1 else None


@functools.partial(jax.jit, static_argnames=("batch_tile",))
def _forward(x, w1, b1, w2, b2, wl1, bl1, wl2, bl2, batch_tile=256):
    B = x.shape[0]
    xb = x.reshape(B, 7, 112).astype(jnp.bfloat16)
    bt = min(batch_tile, B)
    bp = _cdiv(B, bt) * bt
    if bp != B:
        xb = jnp.pad(xb, ((0, bp - B), (0, 0), (0, 0)))

    packed = _build_weights(w1, b1, w2, b2, wl1, bl1, wl2, bl2)
    run = functools.partial(_pallas_forward, batch_tile=bt)
    if _MESH is not None and bp % (_NDEV * bt) == 0:
        wspecs = tuple(P(None, None) for _ in range(6))
        run = shard_map(run, mesh=_MESH,
                        in_specs=(P("d", None, None),) + wspecs,
                        out_specs=P("d", None), check_rep=False)
    out = run(xb, *packed)
    return out[:B]


def kernel(x, w1, b1, w2, b2, wl1, bl1, wl2, bl2):
    return _forward(x, w1, b1, w2, b2, wl1, bl1, wl2, bl2)


# speedup vs baseline: 22.3352x; 1.0727x over previous
import functools

import jax
import jax.numpy as jnp
import numpy as np
from jax.experimental import pallas as pl
from jax.experimental.pallas import tpu as pltpu
from jax.experimental.shard_map import shard_map
from jax.sharding import Mesh, PartitionSpec as P


def _cdiv(a, b):
    return (a + b - 1) // b


def _conv1_pattern():
    m = np.zeros((224, 128, 9), np.float32)
    for kh in range(3):
        for kw in range(3):
            for hp_par in range(2):
                for h_par in range(2):
                    dh = 2 * hp_par + h_par + kh
                    ro, d4 = divmod(dh, 4)
                    for w_par in range(2):
                        for wp in range(13):
                            win = 2 * wp + w_par + kw
                            m[ro * 112 + d4 * 28 + win,
                              ((hp_par * 2 + h_par) * 2 + w_par) * 16 + wp,
                              kh * 3 + kw] = 1.0
    return m


def _conv2_pattern():
    m = np.zeros((80, 32, 9), np.float32)
    for kh in range(3):
        for kw in range(3):
            for h_par2 in range(2):
                dh2 = h_par2 + kh
                for w_par in range(2):
                    for wp2 in range(5):
                        wp = 2 * wp2 + w_par + kw
                        m[dh2 * 16 + wp,
                          (h_par2 * 2 + w_par) * 8 + wp2,
                          kh * 3 + kw] = 1.0
    return m


def _bias1_pattern():
    s = np.zeros((2048, 16), np.float32)
    for blk in range(8):
        for wp in range(13):
            for c in range(16):
                s[(blk * 16 + wp) * 16 + c, c] = 1.0
    return s


def _bias2_pattern():
    s = np.zeros((1024, 32), np.float32)
    for blk in range(4):
        for wp2 in range(5):
            for co in range(32):
                s[(blk * 8 + wp2) * 32 + co, co] = 1.0
    return s


_M1 = _conv1_pattern()
_M2 = _conv2_pattern()
_S1 = _bias1_pattern()
_S2 = _bias2_pattern()
_E240 = np.zeros(2048, np.float32)
_E240[240] = 1.0
_E240K = np.zeros(1280, np.float32)
_E240K[240] = 1.0


def _fused_cnn_kernel(x_ref, w1s_ref, w2s_ref,
                      wl1_ref, bl1_ref, wl2_ref, bl2_ref, o_ref):
    bt = x_ref.shape[0]
    xr = x_ref[...]
    xp = jnp.concatenate(
        [xr, jnp.zeros((bt, 2, 112), jnp.bfloat16)], axis=1)

    lhs1 = jnp.concatenate(
        [xp[:, 0:8, :], xp[:, 1:9, :],
         jnp.ones((bt, 8, 1), jnp.bfloat16)], axis=2)
    lhs1 = lhs1.reshape(bt * 8, 225)
    a1 = jnp.dot(lhs1, w1s_ref[...],
                 preferred_element_type=jnp.float32)
    a1 = jnp.maximum(a1, 0.0).astype(jnp.bfloat16)
    p1e = jnp.maximum(
        jnp.maximum(a1[:, 0:256], a1[:, 256:512]),
        jnp.maximum(a1[:, 512:768], a1[:, 768:1024]))
    p1o = jnp.maximum(
        jnp.maximum(a1[:, 1024:1280], a1[:, 1280:1536]),
        jnp.maximum(a1[:, 1536:1792], a1[:, 1792:2048]))
    p1 = jnp.concatenate([p1e, p1o], axis=1)

    p13 = p1.reshape(bt, 8, 512)
    p1p = jnp.concatenate(
        [p13, jnp.zeros((bt, 2, 512), jnp.bfloat16)], axis=1)
    lhs2 = jnp.concatenate(
        [p1p[:, 0:8, :], p1p[:, 1:9, :], p1p[:, 2:10, 0:256]], axis=2)
    lhs2 = lhs2.reshape(bt * 8, 1280)
    a2 = jnp.dot(lhs2, w2s_ref[...],
                 preferred_element_type=jnp.float32)
    a2 = jnp.maximum(a2, 0.0)
    p2 = jnp.maximum(
        jnp.maximum(a2[:, 0:256], a2[:, 256:512]),
        jnp.maximum(a2[:, 512:768], a2[:, 768:1024]))

    p2u = p2.reshape(bt, 8, 256)
    flat = jnp.concatenate([p2u[:, h, :] for h in range(5)],
                           axis=1).astype(jnp.bfloat16)
    h1 = jnp.dot(flat, wl1_ref[...],
                 preferred_element_type=jnp.float32)
    h1 = jnp.maximum(h1 + bl1_ref[...], 0.0).astype(jnp.bfloat16)
    lg = jnp.dot(h1, wl2_ref[...],
                 preferred_element_type=jnp.float32) + bl2_ref[...]

    mx = jnp.max(lg, axis=1, keepdims=True)
    lse = jnp.log(jnp.sum(jnp.exp(lg - mx), axis=1, keepdims=True)) + mx
    o_ref[...] = lg - lse


def _build_weights(w1, b1, w2, b2, wl1, bl1, wl2, bl2):
    f32 = jnp.float32
    w1s = jnp.einsum("knt,tc->knc", _M1, w1.reshape(9, 16))
    b1row = (_S1 @ b1.reshape(16) + _E240).reshape(1, 2048)
    w1s = jnp.concatenate([w1s.reshape(224, 2048), b1row])
    w1s = w1s.astype(jnp.bfloat16)

    w2s = jnp.einsum("rnt,tio->rino", _M2, w2.reshape(9, 16, 32))
    w2s = w2s.reshape(1280, 1024)
    w2s = w2s + _E240K[:, None] * (_S2 @ b2.reshape(32))[None, :]
    w2s = w2s.astype(jnp.bfloat16)

    wl1r = wl1.reshape(32, 5, 5, 128).transpose(1, 2, 0, 3)
    wl1p = jnp.pad(wl1r, ((0, 0), (0, 3), (0, 0), (0, 0)))
    wl1p = wl1p.reshape(1280, 128).astype(jnp.bfloat16)

    return (w1s, w2s, wl1p,
            bl1.reshape(1, 128).astype(f32),
            wl2.astype(jnp.bfloat16), bl2.reshape(1, 10).astype(f32))


def _pallas_forward(xb, w1s, w2s, wl1p, bl1r, wl2b, bl2r, *, batch_tile):
    bp = xb.shape[0]
    bt = min(batch_tile, bp)
    return pl.pallas_call(
        _fused_cnn_kernel,
        out_shape=jax.ShapeDtypeStruct((bp, 10), jnp.float32),
        grid=(bp // bt,),
        in_specs=[
            pl.BlockSpec((bt, 7, 112), lambda b: (b, 0, 0)),
            pl.BlockSpec((225, 2048), lambda b: (0, 0)),
            pl.BlockSpec((1280, 1024), lambda b: (0, 0)),
            pl.BlockSpec((1280, 128), lambda b: (0, 0)),
            pl.BlockSpec((1, 128), lambda b: (0, 0)),
            pl.BlockSpec((128, 10), lambda b: (0, 0)),
            pl.BlockSpec((1, 10), lambda b: (0, 0)),
        ],
        out_specs=pl.BlockSpec((bt, 10), lambda b: (b, 0)),
        compiler_params=pltpu.CompilerParams(
            dimension_semantics=("parallel",)),
    )(xb, w1s, w2s, wl1p, bl1r, wl2b, bl2r)


_DEVICES = jax.devices()
_NDEV = 2 if len(_DEVICES) >= 2 else 1
_MESH = Mesh(np.array(_DEVICES[:_NDEV]), ("d",)) if _NDEV > 1 else None


@functools.partial(jax.jit, static_argnames=("batch_tile",))
def _forward(x, w1, b1, w2, b2, wl1, bl1, wl2, bl2, batch_tile=256):
    B = x.shape[0]
    xb = x.reshape(B, 7, 112).astype(jnp.bfloat16)
    bt = min(batch_tile, B)
    bp = _cdiv(B, bt) * bt
    if bp != B:
        xb = jnp.pad(xb, ((0, bp - B), (0, 0), (0, 0)))

    packed = _build_weights(w1, b1, w2, b2, wl1, bl1, wl2, bl2)
    run = functools.partial(_pallas_forward, batch_tile=bt)
    if False and _MESH is not None and bp % (_NDEV * bt) == 0:
        wspecs = tuple(P(None, None) for _ in range(6))
        run = shard_map(run, mesh=_MESH,
                        in_specs=(P("d", None, None),) + wspecs,
                        out_specs=P("d", None), check_rep=False)
    out = run(xb, *packed)
    return out[:B]


def kernel(x, w1, b1, w2, b2, wl1, bl1, wl2, bl2):
    return _forward(x, w1, b1, w2, b2, wl1, bl1, wl2, bl2)
